# Initial kernel scaffold; baseline (speedup 1.0000x reference)
import jax
import jax.numpy as jnp
from jax.experimental import pallas as pl


def kernel(sentence, table):
    raise NotImplementedError("write your pallas kernel here")


# speedup vs baseline: 4.6392x; 4.6392x over previous
import functools

import jax
import jax.numpy as jnp
from jax import lax
from jax.experimental import pallas as pl
from jax.experimental.pallas import tpu as pltpu
from jax.experimental.pallas import tpu_sc as plsc

_N_CHARS = 1000
_EMB = 32
_BATCH = 4096
_SEQ = 200
_TOT = _BATCH * _SEQ

_NC = 2
_NS = 16
_NW = _NC * _NS
_PER_W = _TOT // _NW
_CHUNK = 1600
_NCHUNK = _PER_W // _CHUNK


def _make_gather():
    mesh = plsc.VectorSubcoreMesh(core_axis_name="c", subcore_axis_name="s")

    @functools.partial(
        pl.kernel,
        mesh=mesh,
        out_type=jax.ShapeDtypeStruct((_TOT, _EMB), jnp.float32),
        scratch_types=[
            pltpu.VMEM((_CHUNK,), jnp.int32),
            pltpu.VMEM((_CHUNK, _EMB), jnp.float32),
            pltpu.SemaphoreType.DMA,
        ],
        compiler_params=pltpu.CompilerParams(use_tc_tiling_on_sc=False),
    )
    def gather_kernel(table_hbm, idx_hbm, out_hbm, idx_v, rows_v, sem):
        wid = lax.axis_index("s") * _NC + lax.axis_index("c")
        w_base = wid * _PER_W

        def body(i, carry):
            base = pl.multiple_of(w_base + i * _CHUNK, 8)
            pltpu.sync_copy(idx_hbm.at[pl.ds(base, _CHUNK)], idx_v)
            pltpu.async_copy(table_hbm.at[idx_v], rows_v, sem).wait()
            pltpu.sync_copy(rows_v, out_hbm.at[pl.ds(base, _CHUNK)])
            return carry

        lax.fori_loop(0, _NCHUNK, body, 0)

    return gather_kernel


_gather = _make_gather()


@jax.jit
def kernel(sentence, table):
    idx = sentence.reshape(_TOT)
    out = _gather(table, idx)
    return out.reshape(_BATCH, _SEQ, _EMB)


# speedup vs baseline: 4.6542x; 1.0032x over previous
import functools

import jax
import jax.numpy as jnp
from jax import lax
from jax.experimental import pallas as pl
from jax.experimental.pallas import tpu as pltpu
from jax.experimental.pallas import tpu_sc as plsc

_N_CHARS = 1000
_EMB = 32
_BATCH = 4096
_SEQ = 200
_TOT = _BATCH * _SEQ

_NC = 2
_NS = 16
_NW = _NC * _NS
_PER_W = _TOT // _NW
_CHUNK = 1600
_NCHUNK = _PER_W // _CHUNK


def _make_gather():
    mesh = plsc.VectorSubcoreMesh(core_axis_name="c", subcore_axis_name="s")

    @functools.partial(
        pl.kernel,
        mesh=mesh,
        out_type=jax.ShapeDtypeStruct((_TOT, _EMB), jnp.float32),
        scratch_types=[
            pltpu.VMEM((2, _CHUNK), jnp.int32),
            pltpu.VMEM((2, _CHUNK, _EMB), jnp.float32),
            pltpu.SemaphoreType.DMA,
            pltpu.SemaphoreType.DMA,
            pltpu.SemaphoreType.DMA,
            pltpu.SemaphoreType.DMA,
            pltpu.SemaphoreType.DMA,
            pltpu.SemaphoreType.DMA,
        ],
        compiler_params=pltpu.CompilerParams(use_tc_tiling_on_sc=False),
    )
    def gather_kernel(table_hbm, idx_hbm, out_hbm,
                      idx_v, rows_v, si0, si1, sg0, sg1, so0, so1):
        sem_i = (si0, si1)
        sem_g = (sg0, sg1)
        sem_o = (so0, so1)
        wid = lax.axis_index("s") * _NC + lax.axis_index("c")
        w_base = wid * _PER_W

        def start_idx(i):
            b = i % 2
            base = pl.multiple_of(w_base + i * _CHUNK, 8)
            cp = pltpu.make_async_copy(
                idx_hbm.at[pl.ds(base, _CHUNK)], idx_v.at[b], sem_i[b])
            cp.start()
            return cp

        def start_gather(i):
            b = i % 2
            cp = pltpu.make_async_copy(
                table_hbm.at[idx_v.at[b]], rows_v.at[b], sem_g[b])
            cp.start()
            return cp

        def start_out(i):
            b = i % 2
            base = pl.multiple_of(w_base + i * _CHUNK, 8)
            cp = pltpu.make_async_copy(
                rows_v.at[b], out_hbm.at[pl.ds(base, _CHUNK)], sem_o[b])
            cp.start()
            return cp

        idx_cp = {}
        out_cp = {}
        idx_cp[0] = start_idx(0)
        idx_cp[1] = start_idx(1)
        for i in range(_NCHUNK):
            idx_cp[i].wait()
            if i >= 2:
                out_cp[i - 2].wait()
            g = start_gather(i)
            g.wait()
            out_cp[i] = start_out(i)
            if i + 2 < _NCHUNK:
                idx_cp[i + 2] = start_idx(i + 2)
        out_cp[_NCHUNK - 2].wait()
        out_cp[_NCHUNK - 1].wait()

    return gather_kernel


_gather = _make_gather()


@jax.jit
def kernel(sentence, table):
    idx = sentence.reshape(_TOT)
    out = _gather(table, idx)
    return out.reshape(_BATCH, _SEQ, _EMB)


# speedup vs baseline: 6.0203x; 1.2935x over previous
import functools

import jax
import jax.numpy as jnp
from jax import lax
from jax.experimental import pallas as pl
from jax.experimental.pallas import tpu as pltpu
from jax.experimental.pallas import tpu_sc as plsc

_N_CHARS = 1000
_EMB = 32
_BATCH = 4096
_SEQ = 200
_TOT = _BATCH * _SEQ

_NC = 2
_NS = 16
_NW = _NC * _NS
_PER_W = _TOT // _NW
_CHUNK = 1600
_NCHUNK = _PER_W // _CHUNK


def _make_gather():
    mesh = plsc.VectorSubcoreMesh(core_axis_name="c", subcore_axis_name="s")

    @functools.partial(
        pl.kernel,
        mesh=mesh,
        out_type=jax.ShapeDtypeStruct((_TOT, _EMB), jnp.float32),
        scratch_types=[
            pltpu.VMEM((2, _CHUNK), jnp.int32),
            pltpu.VMEM((2, _CHUNK, _EMB), jnp.float32),
            pltpu.VMEM_SHARED((_N_CHARS, _EMB), jnp.float32),
            pltpu.SemaphoreType.DMA,
            pltpu.SemaphoreType.DMA,
            pltpu.SemaphoreType.DMA,
            pltpu.SemaphoreType.DMA,
            pltpu.SemaphoreType.DMA,
            pltpu.SemaphoreType.DMA,
        ],
        compiler_params=pltpu.CompilerParams(use_tc_tiling_on_sc=False),
    )
    def gather_kernel(table_hbm, idx_hbm, out_hbm,
                      idx_v, rows_v, table_sh, si0, si1, sg0, sg1, so0, so1):
        sem_i = (si0, si1)
        sem_g = (sg0, sg1)
        sem_o = (so0, so1)
        sid = lax.axis_index("s")
        wid = sid * _NC + lax.axis_index("c")
        w_base = wid * _PER_W

        @pl.when(sid == 0)
        def _():
            pltpu.sync_copy(table_hbm, table_sh)

        plsc.subcore_barrier()

        def start_idx(i):
            b = i % 2
            base = pl.multiple_of(w_base + i * _CHUNK, 8)
            cp = pltpu.make_async_copy(
                idx_hbm.at[pl.ds(base, _CHUNK)], idx_v.at[b], sem_i[b])
            cp.start()
            return cp

        def start_gather(i):
            b = i % 2
            cp = pltpu.make_async_copy(
                table_sh.at[idx_v.at[b]], rows_v.at[b], sem_g[b])
            cp.start()
            return cp

        def start_out(i):
            b = i % 2
            base = pl.multiple_of(w_base + i * _CHUNK, 8)
            cp = pltpu.make_async_copy(
                rows_v.at[b], out_hbm.at[pl.ds(base, _CHUNK)], sem_o[b])
            cp.start()
            return cp

        idx_cp = {}
        out_cp = {}
        idx_cp[0] = start_idx(0)
        idx_cp[1] = start_idx(1)
        for i in range(_NCHUNK):
            idx_cp[i].wait()
            if i >= 2:
                out_cp[i - 2].wait()
            g = start_gather(i)
            g.wait()
            out_cp[i] = start_out(i)
            if i + 2 < _NCHUNK:
                idx_cp[i + 2] = start_idx(i + 2)
        out_cp[_NCHUNK - 2].wait()
        out_cp[_NCHUNK - 1].wait()

    return gather_kernel


_gather = _make_gather()


@jax.jit
def kernel(sentence, table):
    idx = sentence.reshape(_TOT)
    out = _gather(table, idx)
    return out.reshape(_BATCH, _SEQ, _EMB)
